# baseline (speedup 1.0000x reference)
import functools

import jax
import jax.numpy as jnp
from jax import lax
from jax.experimental import pallas as pl
from jax.experimental.pallas import tpu as pltpu
from jax.experimental.pallas import tpu_sc as plsc

H = 768
EPS = 1e-12

NC, NS = 2, 16
NW = NC * NS
CHUNK = 64


def _sc_gather(table, idx):
    n_tok = idx.shape[0]
    rows_per_w = n_tok // NW
    n_chunks = rows_per_w // CHUNK

    mesh = plsc.VectorSubcoreMesh(core_axis_name="c", subcore_axis_name="s")

    @functools.partial(
        pl.kernel,
        out_type=jax.ShapeDtypeStruct((n_tok, H), jnp.float32),
        mesh=mesh,
        scratch_types=[
            pltpu.VMEM((rows_per_w,), jnp.int32),
            pltpu.VMEM((CHUNK, H), jnp.float32),
            pltpu.VMEM((CHUNK, H), jnp.float32),
            pltpu.SemaphoreType.DMA,
            pltpu.SemaphoreType.DMA,
            pltpu.SemaphoreType.DMA,
            pltpu.SemaphoreType.DMA,
        ],
    )
    def gather_kernel(table_hbm, idx_hbm, out_hbm, idx_v, rows0, rows1,
                      g0, g1, w0, w1):
        wid = lax.axis_index("s") * NC + lax.axis_index("c")
        base = wid * rows_per_w
        pltpu.sync_copy(idx_hbm.at[pl.ds(base, rows_per_w)], idx_v)

        rows = (rows0, rows1)
        gsem = (g0, g1)
        wsem = (w0, w1)
        gcp = [None] * n_chunks
        wcp = [None] * n_chunks

        gcp[0] = pltpu.async_copy(
            table_hbm.at[idx_v.at[pl.ds(0, CHUNK)]], rows[0], gsem[0])
        for c in range(n_chunks):
            b = c & 1
            gcp[c].wait()
            if c + 1 < n_chunks:
                nb = (c + 1) & 1
                if c >= 1:
                    wcp[c - 1].wait()
                gcp[c + 1] = pltpu.async_copy(
                    table_hbm.at[idx_v.at[pl.ds((c + 1) * CHUNK, CHUNK)]],
                    rows[nb], gsem[nb])
            wcp[c] = pltpu.async_copy(
                rows[b], out_hbm.at[pl.ds(base + c * CHUNK, CHUNK)], wsem[b])
        wcp[n_chunks - 2].wait()
        wcp[n_chunks - 1].wait()

    return gather_kernel(table, idx)


def _tc_fused(gathered, tt, tt_w, tt_b, gamma, beta):
    n_tok = gathered.shape[0]
    bt = 1024

    def body(g_ref, t_ref, w_ref, b_ref, gam_ref, bet_ref, o_ref):
        ttl = jnp.dot(t_ref[...], w_ref[...],
                      preferred_element_type=jnp.float32,
                      precision=lax.Precision.HIGHEST)
        x = g_ref[...] + ttl + b_ref[...]
        mu = jnp.mean(x, axis=-1, keepdims=True)
        d = x - mu
        var = jnp.mean(d * d, axis=-1, keepdims=True)
        o_ref[...] = d * lax.rsqrt(var + EPS) * gam_ref[...] + bet_ref[...]

    return pl.pallas_call(
        body,
        grid=(n_tok // bt,),
        in_specs=[
            pl.BlockSpec((bt, H), lambda i: (i, 0)),
            pl.BlockSpec((bt, 3), lambda i: (i, 0)),
            pl.BlockSpec((3, H), lambda i: (0, 0)),
            pl.BlockSpec((1, H), lambda i: (0, 0)),
            pl.BlockSpec((1, H), lambda i: (0, 0)),
            pl.BlockSpec((1, H), lambda i: (0, 0)),
        ],
        out_specs=pl.BlockSpec((bt, H), lambda i: (i, 0)),
        out_shape=jax.ShapeDtypeStruct((n_tok, H), jnp.float32),
        compiler_params=pltpu.CompilerParams(
            dimension_semantics=("parallel",)),
    )(gathered, tt, tt_w, tt_b, gamma, beta)


@jax.jit
def kernel(input_ids, token_type_ids, word_table, tt_w, tt_b, ln_gamma, ln_beta):
    b, s = input_ids.shape
    idx = input_ids.reshape(-1).astype(jnp.int32)
    tt = token_type_ids.reshape(-1, 3)

    gathered = _sc_gather(word_table, idx)
    out = _tc_fused(
        gathered,
        tt,
        tt_w,
        tt_b.reshape(1, H),
        ln_gamma.reshape(1, H),
        ln_beta.reshape(1, H),
    )
    return out.reshape(b, s, H)


# speedup vs baseline: 1.1686x; 1.1686x over previous
import functools

import jax
import jax.numpy as jnp
from jax import lax
from jax.experimental import pallas as pl
from jax.experimental.pallas import tpu as pltpu
from jax.experimental.pallas import tpu_sc as plsc

H = 768
EPS = 1e-12

NC, NS = 2, 16
NW = NC * NS
CHUNK = 64


def _sc_gather(table, idx):
    n_tok = idx.shape[0]
    rows_per_w = n_tok // NW
    n_chunks = rows_per_w // CHUNK

    mesh = plsc.VectorSubcoreMesh(core_axis_name="c", subcore_axis_name="s")

    @functools.partial(
        pl.kernel,
        out_type=jax.ShapeDtypeStruct((n_tok, H), jnp.float32),
        mesh=mesh,
        scratch_types=[
            pltpu.VMEM((rows_per_w,), jnp.int32),
            pltpu.VMEM((CHUNK, H), jnp.float32),
            pltpu.VMEM((CHUNK, H), jnp.float32),
            pltpu.SemaphoreType.DMA,
            pltpu.SemaphoreType.DMA,
            pltpu.SemaphoreType.DMA,
            pltpu.SemaphoreType.DMA,
        ],
    )
    def gather_kernel(table_hbm, idx_hbm, out_hbm, idx_v, rows0, rows1,
                      g0, g1, w0, w1):
        wid = lax.axis_index("s") * NC + lax.axis_index("c")
        base = wid * rows_per_w
        pltpu.sync_copy(idx_hbm.at[pl.ds(base, rows_per_w)], idx_v)

        rows = (rows0, rows1)
        gsem = (g0, g1)
        wsem = (w0, w1)
        gcp = [None] * n_chunks
        wcp = [None] * n_chunks

        gcp[0] = pltpu.async_copy(
            table_hbm.at[idx_v.at[pl.ds(0, CHUNK)]], rows[0], gsem[0])
        for c in range(n_chunks):
            b = c & 1
            gcp[c].wait()
            if c + 1 < n_chunks:
                nb = (c + 1) & 1
                if c >= 1:
                    wcp[c - 1].wait()
                gcp[c + 1] = pltpu.async_copy(
                    table_hbm.at[idx_v.at[pl.ds((c + 1) * CHUNK, CHUNK)]],
                    rows[nb], gsem[nb])
            wcp[c] = pltpu.async_copy(
                rows[b], out_hbm.at[pl.ds(base + c * CHUNK, CHUNK)], wsem[b])
        wcp[n_chunks - 2].wait()
        wcp[n_chunks - 1].wait()

    return gather_kernel(table, idx)


def _tc_fused(gathered, tt, tt_w, tt_b, gamma, beta):
    n_tok = gathered.shape[0]
    bt = 1024

    def body(g_ref, t_ref, w_ref, b_ref, gam_ref, bet_ref, o_ref):
        ttl = jnp.dot(t_ref[...], w_ref[...],
                      preferred_element_type=jnp.float32,
                      precision=lax.Precision.DEFAULT)
        x = g_ref[...] + ttl + b_ref[...]
        mu = jnp.mean(x, axis=-1, keepdims=True)
        d = x - mu
        var = jnp.mean(d * d, axis=-1, keepdims=True)
        o_ref[...] = d * lax.rsqrt(var + EPS) * gam_ref[...] + bet_ref[...]

    return pl.pallas_call(
        body,
        grid=(n_tok // bt,),
        in_specs=[
            pl.BlockSpec((bt, H), lambda i: (i, 0)),
            pl.BlockSpec((bt, 3), lambda i: (i, 0)),
            pl.BlockSpec((3, H), lambda i: (0, 0)),
            pl.BlockSpec((1, H), lambda i: (0, 0)),
            pl.BlockSpec((1, H), lambda i: (0, 0)),
            pl.BlockSpec((1, H), lambda i: (0, 0)),
        ],
        out_specs=pl.BlockSpec((bt, H), lambda i: (i, 0)),
        out_shape=jax.ShapeDtypeStruct((n_tok, H), jnp.float32),
        compiler_params=pltpu.CompilerParams(
            dimension_semantics=("parallel",)),
    )(gathered, tt, tt_w, tt_b, gamma, beta)


@jax.jit
def kernel(input_ids, token_type_ids, word_table, tt_w, tt_b, ln_gamma, ln_beta):
    b, s = input_ids.shape
    idx = input_ids.reshape(-1).astype(jnp.int32)
    tt = token_type_ids.reshape(-1, 3)

    gathered = _sc_gather(word_table, idx)
    out = _tc_fused(
        gathered,
        tt,
        tt_w,
        tt_b.reshape(1, H),
        ln_gamma.reshape(1, H),
        ln_beta.reshape(1, H),
    )
    return out.reshape(b, s, H)
